# Initial kernel scaffold; baseline (speedup 1.0000x reference)
import jax
import jax.numpy as jnp
from jax.experimental import pallas as pl


def kernel(X, Edge_index, Edge_weight, Batching, conv1_w, conv1_b, conv2_w, conv2_b, phi_w1, phi_b1, phi_w2, phi_b2, phi_w3, phi_b3, th_w1, th_b1, th_w2, th_b2, th_w3, th_b3):
    raise NotImplementedError("write your pallas kernel here")


# speedup vs baseline: 1.1680x; 1.1680x over previous
import functools

import jax
import jax.numpy as jnp
from jax import lax
from jax.experimental import pallas as pl
from jax.experimental.pallas import tpu as pltpu

N = 100000
E = 3200000
NUM_GRAPHS = 64
K = 3
IN_CH = 3
GCN = 16
DENSE = 16
NEG_SLOPE = 0.01

BLK = 10000


def _lrelu(x):
    return jnp.where(x >= 0, x, NEG_SLOPE * x)


def _disg0_body(deg_ref, xp_ref, dis_ref, g0_ref):
    dis = lax.rsqrt(deg_ref[...] + 1.0)
    dis_ref[...] = dis
    g0_ref[...] = xp_ref[...] * dis


def _hop_body(s_ref, hp_ref, dis_ref, h_ref, g_ref):
    dis = dis_ref[...]
    h = dis * s_ref[...] + (dis * dis) * hp_ref[...]
    h_ref[...] = h
    g_ref[...] = h * dis


def _conv_body(cat_ref, w_ref, b_ref, dis_ref, out_ref, g_ref):
    o = jnp.dot(cat_ref[...], w_ref[...], preferred_element_type=jnp.float32)
    o = _lrelu(o + b_ref[...])
    out_ref[...] = o
    g_ref[...] = o * dis_ref[...]


def _pool_heads_body(out2_ref, bat_ref,
                     pw1_ref, pb1_ref, pw2_ref, pb2_ref, pw3_ref, pb3_ref,
                     tw1_ref, tb1_ref, tw2_ref, tb2_ref, tw3_ref, tb3_ref,
                     res_ref, sums, cnts):
    i = pl.program_id(0)

    @pl.when(i == 0)
    def _():
        sums[...] = jnp.zeros_like(sums)
        cnts[...] = jnp.zeros_like(cnts)

    onehot = (bat_ref[...] == lax.broadcasted_iota(
        jnp.int32, (1, NUM_GRAPHS), 1)).astype(jnp.float32)
    dn = (((0,), (0,)), ((), ()))
    sums[...] += lax.dot_general(onehot, out2_ref[...], dn,
                                 preferred_element_type=jnp.float32)
    cnts[...] += lax.dot_general(onehot,
                                 jnp.ones((onehot.shape[0], 1), jnp.float32),
                                 dn, preferred_element_type=jnp.float32)

    @pl.when(i == pl.num_programs(0) - 1)
    def _():
        pooled = sums[...] / jnp.maximum(cnts[...], 1.0)
        p = _lrelu(jnp.dot(pooled, pw1_ref[...],
                           preferred_element_type=jnp.float32) + pb1_ref[...])
        p = _lrelu(jnp.dot(p, pw2_ref[...],
                           preferred_element_type=jnp.float32) + pb2_ref[...])
        p = jnp.dot(p, pw3_ref[...],
                    preferred_element_type=jnp.float32) + pb3_ref[...]
        t = _lrelu(jnp.dot(pooled, tw1_ref[...],
                           preferred_element_type=jnp.float32) + tb1_ref[...])
        t = _lrelu(jnp.dot(t, tw2_ref[...],
                           preferred_element_type=jnp.float32) + tb2_ref[...])
        t = jnp.dot(t, tw3_ref[...],
                    preferred_element_type=jnp.float32) + tb3_ref[...]
        res_ref[...] = jnp.concatenate([p, t], axis=1)


def _row_spec(width):
    return pl.BlockSpec((BLK, width), lambda i: (i, 0))


def _full_spec(shape):
    return pl.BlockSpec(shape, lambda i: (0, 0))


def _disg0(deg, xp):
    return pl.pallas_call(
        _disg0_body,
        grid=(N // BLK,),
        in_specs=[_row_spec(1), _row_spec(GCN)],
        out_specs=[_row_spec(1), _row_spec(GCN)],
        out_shape=[jax.ShapeDtypeStruct((N, 1), jnp.float32),
                   jax.ShapeDtypeStruct((N, GCN), jnp.float32)],
    )(deg, xp)


def _hop_combine(s, hp, dis):
    return pl.pallas_call(
        _hop_body,
        grid=(N // BLK,),
        in_specs=[_row_spec(GCN), _row_spec(GCN), _row_spec(1)],
        out_specs=[_row_spec(GCN), _row_spec(GCN)],
        out_shape=[jax.ShapeDtypeStruct((N, GCN), jnp.float32),
                   jax.ShapeDtypeStruct((N, GCN), jnp.float32)],
    )(s, hp, dis)


def _conv_out(cat, wcat, b, dis):
    return pl.pallas_call(
        _conv_body,
        grid=(N // BLK,),
        in_specs=[_row_spec(4 * GCN), _full_spec((4 * GCN, GCN)),
                  _full_spec((1, GCN)), _row_spec(1)],
        out_specs=[_row_spec(GCN), _row_spec(GCN)],
        out_shape=[jax.ShapeDtypeStruct((N, GCN), jnp.float32),
                   jax.ShapeDtypeStruct((N, GCN), jnp.float32)],
    )(cat, wcat, b.reshape(1, GCN), dis)


def _pool_heads(out2, bat, pw1, pb1, pw2, pb2, pw3, pb3,
                tw1, tb1, tw2, tb2, tw3, tb3):
    full = [_full_spec((GCN, DENSE)), _full_spec((1, DENSE)),
            _full_spec((DENSE, DENSE)), _full_spec((1, DENSE)),
            _full_spec((DENSE, 2)), _full_spec((1, 2))] * 2
    return pl.pallas_call(
        _pool_heads_body,
        grid=(N // BLK,),
        in_specs=[_row_spec(GCN), _row_spec(1)] + full,
        out_specs=pl.BlockSpec((NUM_GRAPHS, 4), lambda i: (0, 0)),
        out_shape=jax.ShapeDtypeStruct((NUM_GRAPHS, 4), jnp.float32),
        scratch_shapes=[pltpu.VMEM((NUM_GRAPHS, GCN), jnp.float32),
                        pltpu.VMEM((NUM_GRAPHS, 1), jnp.float32)],
    )(out2, bat,
      pw1.T, pb1.reshape(1, -1), pw2.T, pb2.reshape(1, -1),
      pw3.T, pb3.reshape(1, -1),
      tw1.T, tb1.reshape(1, -1), tw2.T, tb2.reshape(1, -1),
      tw3.T, tb3.reshape(1, -1))


def _edge_deg(dst, ew):
    return jax.ops.segment_sum(ew, dst, num_segments=N)


def _edge_prop(src, dst, ew, g):
    return jax.ops.segment_sum(ew[:, None] * g[src], dst, num_segments=N)


def _wcat(conv_w, in_ch):
    blocks = []
    for t in range(K + 1):
        wt = conv_w[t].T
        if in_ch < GCN:
            wt = jnp.pad(wt, ((0, GCN - in_ch), (0, 0)))
        blocks.append(wt)
    return jnp.concatenate(blocks, axis=0)


def kernel(X, Edge_index, Edge_weight, Batching, conv1_w, conv1_b, conv2_w,
           conv2_b, phi_w1, phi_b1, phi_w2, phi_b2, phi_w3, phi_b3,
           th_w1, th_b1, th_w2, th_b2, th_w3, th_b3):
    src = Edge_index[0]
    dst = Edge_index[1]

    deg = _edge_deg(dst, Edge_weight).reshape(N, 1)
    xp = jnp.pad(X, ((0, 0), (0, GCN - IN_CH)))
    dis, g = _disg0(deg, xp)

    h = xp
    hops = [xp]
    for _ in range(K):
        s = _edge_prop(src, dst, Edge_weight, g)
        h, g = _hop_combine(s, h, dis)
        hops.append(h)
    out1, g = _conv_out(jnp.concatenate(hops, axis=1),
                        _wcat(conv1_w, IN_CH), conv1_b, dis)

    h = out1
    hops = [out1]
    for _ in range(K):
        s = _edge_prop(src, dst, Edge_weight, g)
        h, g = _hop_combine(s, h, dis)
        hops.append(h)
    out2, _ = _conv_out(jnp.concatenate(hops, axis=1),
                        _wcat(conv2_w, GCN), conv2_b, dis)

    return _pool_heads(out2, Batching.reshape(N, 1),
                       phi_w1, phi_b1, phi_w2, phi_b2, phi_w3, phi_b3,
                       th_w1, th_b1, th_w2, th_b2, th_w3, th_b3)


# speedup vs baseline: 26.5735x; 22.7507x over previous
import functools

import jax
import jax.numpy as jnp
from jax import lax
from jax.experimental import pallas as pl
from jax.experimental.pallas import tpu as pltpu
from jax.experimental.pallas import tpu_sc as plsc

N = 100000
E = 3200000
NUM_GRAPHS = 64
K = 3
IN_CH = 3
GCN = 16
DENSE = 16
NEG_SLOPE = 0.01

BLK = 4000


def _lrelu(x):
    return jnp.where(x >= 0, x, NEG_SLOPE * x)


def _disg0_body(d0_ref, d1_ref, xp_ref, dis_ref, g0_ref):
    dis = lax.rsqrt(d0_ref[...] + d1_ref[...] + 1.0)
    dis_ref[...] = dis
    g0_ref[...] = xp_ref[...] * dis


def _hop_body(s0_ref, s1_ref, hp_ref, dis_ref, h_ref, g_ref):
    dis = dis_ref[...]
    h = dis * (s0_ref[...] + s1_ref[...]) + (dis * dis) * hp_ref[...]
    h_ref[...] = h
    g_ref[...] = h * dis


def _conv_body(cat_ref, w_ref, b_ref, dis_ref, out_ref, g_ref):
    o = jnp.dot(cat_ref[...], w_ref[...], preferred_element_type=jnp.float32)
    o = _lrelu(o + b_ref[...])
    out_ref[...] = o
    g_ref[...] = o * dis_ref[...]


def _pool_heads_body(out2_ref, bat_ref,
                     pw1_ref, pb1_ref, pw2_ref, pb2_ref, pw3_ref, pb3_ref,
                     tw1_ref, tb1_ref, tw2_ref, tb2_ref, tw3_ref, tb3_ref,
                     res_ref, sums, cnts):
    i = pl.program_id(0)

    @pl.when(i == 0)
    def _():
        sums[...] = jnp.zeros_like(sums)
        cnts[...] = jnp.zeros_like(cnts)

    onehot = (bat_ref[...] == lax.broadcasted_iota(
        jnp.int32, (1, NUM_GRAPHS), 1)).astype(jnp.float32)
    dn = (((0,), (0,)), ((), ()))
    sums[...] += lax.dot_general(onehot, out2_ref[...], dn,
                                 preferred_element_type=jnp.float32)
    cnts[...] += lax.dot_general(onehot,
                                 jnp.ones((onehot.shape[0], 1), jnp.float32),
                                 dn, preferred_element_type=jnp.float32)

    @pl.when(i == pl.num_programs(0) - 1)
    def _():
        pooled = sums[...] / jnp.maximum(cnts[...], 1.0)
        p = _lrelu(jnp.dot(pooled, pw1_ref[...],
                           preferred_element_type=jnp.float32) + pb1_ref[...])
        p = _lrelu(jnp.dot(p, pw2_ref[...],
                           preferred_element_type=jnp.float32) + pb2_ref[...])
        p = jnp.dot(p, pw3_ref[...],
                    preferred_element_type=jnp.float32) + pb3_ref[...]
        t = _lrelu(jnp.dot(pooled, tw1_ref[...],
                           preferred_element_type=jnp.float32) + tb1_ref[...])
        t = _lrelu(jnp.dot(t, tw2_ref[...],
                           preferred_element_type=jnp.float32) + tb2_ref[...])
        t = jnp.dot(t, tw3_ref[...],
                    preferred_element_type=jnp.float32) + tb3_ref[...]
        res_ref[...] = jnp.concatenate([p, t], axis=1)


def _row_spec(width):
    return pl.BlockSpec((BLK, width), lambda i: (i, 0))


def _full_spec(shape):
    return pl.BlockSpec(shape, lambda i: (0, 0))


def _disg0(d0, d1, xp):
    return pl.pallas_call(
        _disg0_body,
        grid=(N // BLK,),
        in_specs=[_row_spec(1), _row_spec(1), _row_spec(GCN)],
        out_specs=[_row_spec(1), _row_spec(GCN)],
        out_shape=[jax.ShapeDtypeStruct((N, 1), jnp.float32),
                   jax.ShapeDtypeStruct((N, GCN), jnp.float32)],
    )(d0, d1, xp)


def _hop_combine(s0, s1, hp, dis):
    return pl.pallas_call(
        _hop_body,
        grid=(N // BLK,),
        in_specs=[_row_spec(GCN), _row_spec(GCN), _row_spec(GCN), _row_spec(1)],
        out_specs=[_row_spec(GCN), _row_spec(GCN)],
        out_shape=[jax.ShapeDtypeStruct((N, GCN), jnp.float32),
                   jax.ShapeDtypeStruct((N, GCN), jnp.float32)],
    )(s0, s1, hp, dis)


def _conv_out(cat, wcat, b, dis):
    return pl.pallas_call(
        _conv_body,
        grid=(N // BLK,),
        in_specs=[_row_spec(4 * GCN), _full_spec((4 * GCN, GCN)),
                  _full_spec((1, GCN)), _row_spec(1)],
        out_specs=[_row_spec(GCN), _row_spec(GCN)],
        out_shape=[jax.ShapeDtypeStruct((N, GCN), jnp.float32),
                   jax.ShapeDtypeStruct((N, GCN), jnp.float32)],
    )(cat, wcat, b.reshape(1, GCN), dis)


def _pool_heads(out2, bat, pw1, pb1, pw2, pb2, pw3, pb3,
                tw1, tb1, tw2, tb2, tw3, tb3):
    full = [_full_spec((GCN, DENSE)), _full_spec((1, DENSE)),
            _full_spec((DENSE, DENSE)), _full_spec((1, DENSE)),
            _full_spec((DENSE, 2)), _full_spec((1, 2))] * 2
    return pl.pallas_call(
        _pool_heads_body,
        grid=(N // BLK,),
        in_specs=[_row_spec(GCN), _row_spec(1)] + full,
        out_specs=pl.BlockSpec((NUM_GRAPHS, 4), lambda i: (0, 0)),
        out_shape=jax.ShapeDtypeStruct((NUM_GRAPHS, 4), jnp.float32),
        scratch_shapes=[pltpu.VMEM((NUM_GRAPHS, GCN), jnp.float32),
                        pltpu.VMEM((NUM_GRAPHS, 1), jnp.float32)],
    )(out2, bat,
      pw1.T, pb1.reshape(1, -1), pw2.T, pb2.reshape(1, -1),
      pw3.T, pb3.reshape(1, -1),
      tw1.T, tb1.reshape(1, -1), tw2.T, tb2.reshape(1, -1),
      tw3.T, tb3.reshape(1, -1))


NC = 2
NS = 16
RW = 128
SUP = 512
RPS = SUP // RW
NSUP = 198
NB = 3
EPT = NSUP * SUP
EP = EPT * NC * NS
NPAD = 100096
NPT = NPAD // NS


def _sc_mesh():
    return plsc.VectorSubcoreMesh(core_axis_name="c", subcore_axis_name="s")


def _prop_sc(g, src2, dst2, ew, gather=True):
    @functools.partial(
        pl.kernel, mesh=_sc_mesh(),
        compiler_params=pltpu.CompilerParams(use_tc_tiling_on_sc=False),
        out_type=jax.ShapeDtypeStruct((NC, NPAD, GCN), jnp.float32),
        scratch_types=[
            pltpu.VMEM((NB, RPS, RW), jnp.int32),
            pltpu.VMEM((NB, RPS, RW), jnp.int32),
            pltpu.VMEM((NB, SUP), jnp.float32),
            pltpu.VMEM((NB, SUP, GCN), jnp.float32),
            pltpu.VMEM_SHARED((NPAD, GCN), jnp.float32),
        ] + [pltpu.SemaphoreType.DMA] * (3 * NB),
    )
    def k(g_hbm, src_hbm, dst_hbm, ew_hbm, z_hbm, out_hbm,
          src_v, dst_v, ew_v, rows_v, acc_sh,
          sg0, sg1, sg2, ss0, ss1, ss2, sl0, sl1, sl2):
        SG = (sg0, sg1, sg2)
        SS = (ss0, ss1, ss2)
        SL = (sl0, sl1, sl2)
        c = lax.axis_index("c")
        s = lax.axis_index("s")
        wid = c * NS + s
        rowb = wid * (EPT // RW)
        eb = wid * EPT
        rbase = s * NPT

        pltpu.sync_copy(z_hbm.at[pl.ds(rbase, NPT)],
                        acc_sh.at[pl.ds(rbase, NPT)])
        plsc.subcore_barrier()

        def produce(b, t, drain):
            if drain:
                @pl.when(t >= NB)
                def _():
                    pltpu.make_async_copy(
                        rows_v.at[b], acc_sh.at[pl.ds(0, SUP)], SS[b]).wait()
            if gather:
                l1 = pltpu.async_copy(
                    src_hbm.at[pl.ds(rowb + t * RPS, RPS)], src_v.at[b], SL[b])
                l1.wait()
            l2 = pltpu.async_copy(
                dst_hbm.at[pl.ds(rowb + t * RPS, RPS)], dst_v.at[b], SL[b])
            l3 = pltpu.async_copy(
                ew_hbm.at[pl.ds(eb + t * SUP, SUP)], ew_v.at[b], SL[b])
            l2.wait()
            l3.wait()
            if gather:
                for j in range(RPS):
                    pltpu.async_copy(g_hbm.at[src_v.at[b].at[j]],
                                     rows_v.at[b].at[pl.ds(j * RW, RW)], SG[b])

        def consume(b, t):
            if gather:
                pltpu.make_async_copy(
                    g_hbm.at[pl.ds(0, SUP)], rows_v.at[b], SG[b]).wait()

                def scale(j, carry):
                    w16 = ew_v[b, pl.ds(j * 16, 16)]
                    for l in range(16):
                        r = j * 16 + l
                        rows_v[b, r] = rows_v[b, r] * w16[l]
                    return carry
            else:
                ones = jnp.ones((16,), jnp.float32)

                def scale(j, carry):
                    w16 = ew_v[b, pl.ds(j * 16, 16)]
                    for l in range(16):
                        rows_v[b, j * 16 + l] = ones * w16[l]
                    return carry
            lax.fori_loop(0, SUP // 16, scale, 0)

            for j in range(RPS):
                pltpu.async_copy(rows_v.at[b].at[pl.ds(j * RW, RW)],
                                 acc_sh.at[dst_v.at[b].at[j]], SS[b], add=True)

        produce(0, 0, False)
        produce(1, 1, False)

        def outer(i, carry):
            for kk in range(NB):
                t = i * NB + kk
                consume(kk, t)
                bn = (kk + 2) % NB

                @pl.when(t + 2 < NSUP)
                def _():
                    produce(bn, t + 2, True)
            return carry
        lax.fori_loop(0, NSUP // NB, outer, 0)

        for b in range(NB):
            pltpu.make_async_copy(
                rows_v.at[b], acc_sh.at[pl.ds(0, SUP)], SS[b]).wait()
        plsc.subcore_barrier()

        pltpu.sync_copy(acc_sh.at[pl.ds(rbase, NPT)],
                        out_hbm.at[c, pl.ds(rbase, NPT)])

    return k(g, src2, dst2, ew, jnp.zeros((NPAD, GCN), jnp.float32))


def _wcat(conv_w, in_ch):
    blocks = []
    for t in range(K + 1):
        wt = conv_w[t].T
        if in_ch < GCN:
            wt = jnp.pad(wt, ((0, GCN - in_ch), (0, 0)))
        blocks.append(wt)
    return jnp.concatenate(blocks, axis=0)


def kernel(X, Edge_index, Edge_weight, Batching, conv1_w, conv1_b, conv2_w,
           conv2_b, phi_w1, phi_b1, phi_w2, phi_b2, phi_w3, phi_b3,
           th_w1, th_b1, th_w2, th_b2, th_w3, th_b3):
    pad = EP - E
    srcp = jnp.concatenate([Edge_index[0], jnp.zeros((pad,), jnp.int32)])
    dstp = jnp.concatenate([Edge_index[1], jnp.zeros((pad,), jnp.int32)])
    ewp = jnp.concatenate([Edge_weight, jnp.zeros((pad,), jnp.float32)])
    src2 = srcp.reshape(EP // RW, RW)
    dst2 = dstp.reshape(EP // RW, RW)

    degp = _prop_sc(jnp.zeros((16, GCN), jnp.float32), dst2, dst2, ewp,
                    gather=False)
    xp = jnp.pad(X, ((0, 0), (0, GCN - IN_CH)))
    dis, g = _disg0(degp[0, :N, :1], degp[1, :N, :1], xp)

    h = xp
    hops = [xp]
    for _ in range(K):
        sp = _prop_sc(g, src2, dst2, ewp)
        h, g = _hop_combine(sp[0, :N], sp[1, :N], h, dis)
        hops.append(h)
    out1, g = _conv_out(jnp.concatenate(hops, axis=1),
                        _wcat(conv1_w, IN_CH), conv1_b, dis)

    h = out1
    hops = [out1]
    for _ in range(K):
        sp = _prop_sc(g, src2, dst2, ewp)
        h, g = _hop_combine(sp[0, :N], sp[1, :N], h, dis)
        hops.append(h)
    out2, _ = _conv_out(jnp.concatenate(hops, axis=1),
                        _wcat(conv2_w, GCN), conv2_b, dis)

    return _pool_heads(out2, Batching.reshape(N, 1),
                       phi_w1, phi_b1, phi_w2, phi_b2, phi_w3, phi_b3,
                       th_w1, th_b1, th_w2, th_b2, th_w3, th_b3)
